# Initial kernel scaffold; baseline (speedup 1.0000x reference)
import jax
import jax.numpy as jnp
from jax.experimental import pallas as pl


def kernel(x0, x1, x2, bp_0, bp_1, bp_2, values):
    raise NotImplementedError("write your pallas kernel here")


# speedup vs baseline: 2.8548x; 2.8548x over previous
import functools

import jax
import jax.numpy as jnp
from jax import lax
from jax.experimental import pallas as pl
from jax.experimental.pallas import tpu as pltpu
from jax.experimental.pallas import tpu_sc as plsc

N = 1048576
G = 128
L = 16

_info = plsc.get_sparse_core_info()
_NC, _NS = _info.num_cores, _info.num_subcores
NW = _NC * _NS
PTS = N // NW
CHUNK = 4096
NCHUNK = PTS // CHUNK


def _searchsorted(bp_ref, x, bp_last):
    c = jnp.zeros((L,), jnp.int32)
    for step in (64, 32, 16, 8, 4, 2, 1):
        probe = c + (step - 1)
        g = plsc.load_gather(bp_ref, [probe])
        c = jnp.where(g <= x, c + step, c)
    return jnp.where(bp_last <= x, G - 1, jnp.maximum(c - 1, 0))


def _body(x0_hbm, x1_hbm, x2_hbm, bp0_hbm, bp1_hbm, bp2_hbm, vals_hbm,
          out_hbm, x0_v, x1_v, x2_v, bp0_v, bp1_v, bp2_v, idx_v, res_v, sem):
    wid = lax.axis_index("s") * _NC + lax.axis_index("c")
    base = wid * PTS

    pltpu.sync_copy(bp0_hbm, bp0_v)
    pltpu.sync_copy(bp1_hbm, bp1_v)
    pltpu.sync_copy(bp2_hbm, bp2_v)
    last = jnp.full((L,), G - 1, jnp.int32)
    bp0_last = plsc.load_gather(bp0_v, [last])
    bp1_last = plsc.load_gather(bp1_v, [last])
    bp2_last = plsc.load_gather(bp2_v, [last])

    def chunk_body(ci, _):
        off = base + ci * CHUNK
        pltpu.sync_copy(x0_hbm.at[pl.ds(off, CHUNK)], x0_v)
        pltpu.sync_copy(x1_hbm.at[pl.ds(off, CHUNK)], x1_v)
        pltpu.sync_copy(x2_hbm.at[pl.ds(off, CHUNK)], x2_v)

        def vec_body(vi, _):
            s = vi * L
            x0 = x0_v[pl.ds(s, L)]
            x1 = x1_v[pl.ds(s, L)]
            x2 = x2_v[pl.ds(s, L)]
            i0 = _searchsorted(bp0_v, x0, bp0_last)
            i1 = _searchsorted(bp1_v, x1, bp1_last)
            i2 = _searchsorted(bp2_v, x2, bp2_last)
            idx_v[pl.ds(s, L)] = (i0 * G + i1) * G + i2
            return 0

        lax.fori_loop(0, CHUNK // L, vec_body, 0)
        pltpu.async_copy(vals_hbm.at[idx_v], res_v, sem).wait()
        pltpu.sync_copy(res_v, out_hbm.at[pl.ds(off, CHUNK)])
        return 0

    lax.fori_loop(0, NCHUNK, chunk_body, 0)


@jax.jit
def _run(x0, x1, x2, bp_0, bp_1, bp_2, values_flat):
    mesh = plsc.VectorSubcoreMesh(core_axis_name="c", subcore_axis_name="s")
    k = pl.kernel(
        _body,
        out_type=jax.ShapeDtypeStruct((N,), jnp.float32),
        mesh=mesh,
        compiler_params=pltpu.CompilerParams(needs_layout_passes=False),
        scratch_types=[
            pltpu.VMEM((CHUNK,), jnp.float32),
            pltpu.VMEM((CHUNK,), jnp.float32),
            pltpu.VMEM((CHUNK,), jnp.float32),
            pltpu.VMEM((G,), jnp.float32),
            pltpu.VMEM((G,), jnp.float32),
            pltpu.VMEM((G,), jnp.float32),
            pltpu.VMEM((CHUNK,), jnp.int32),
            pltpu.VMEM((CHUNK,), jnp.float32),
            pltpu.SemaphoreType.DMA,
        ],
    )
    return k(x0, x1, x2, bp_0, bp_1, bp_2, values_flat)


def kernel(x0, x1, x2, bp_0, bp_1, bp_2, values):
    return _run(x0, x1, x2, bp_0, bp_1, bp_2, values.reshape(-1))


# speedup vs baseline: 3.6495x; 1.2784x over previous
import jax
import jax.numpy as jnp
from jax import lax
from jax.experimental import pallas as pl
from jax.experimental.pallas import tpu as pltpu
from jax.experimental.pallas import tpu_sc as plsc

N = 1048576
G = 128
L = 16

_info = plsc.get_sparse_core_info()
_NC, _NS = _info.num_cores, _info.num_subcores
NW = _NC * _NS
PTS = N // NW
CHUNK = 8192
NCHUNK = PTS // CHUNK
UNROLL = 4


def _searchsorted(bp_ref, x, pre):
    bp31, bp63, bp95, bp127 = pre
    c64 = jnp.where(bp63 <= x, 64, 0)
    lvl1 = jnp.where(bp63 <= x, bp95, bp31)
    c = jnp.where(lvl1 <= x, c64 + 32, c64)
    for step in (16, 8, 4, 2, 1):
        probe = c + (step - 1)
        g = plsc.load_gather(bp_ref, [probe])
        c = jnp.where(g <= x, c + step, c)
    return jnp.where(bp127 <= x, G - 1, jnp.maximum(c - 1, 0))


def _preload(bp_v):
    def bcast(i):
        return plsc.load_gather(bp_v, [jnp.full((L,), i, jnp.int32)])
    return bcast(31), bcast(63), bcast(95), bcast(127)


def _body(x0_hbm, x1_hbm, x2_hbm, bp0_hbm, bp1_hbm, bp2_hbm, vals_hbm,
          out_hbm, x0_v, x1_v, x2_v, bp0_v, bp1_v, bp2_v, idx_v, res_v, sem):
    wid = lax.axis_index("s") * _NC + lax.axis_index("c")
    base = wid * PTS

    pltpu.sync_copy(bp0_hbm, bp0_v)
    pltpu.sync_copy(bp1_hbm, bp1_v)
    pltpu.sync_copy(bp2_hbm, bp2_v)
    pre0 = _preload(bp0_v)
    pre1 = _preload(bp1_v)
    pre2 = _preload(bp2_v)

    def chunk_body(ci, _):
        off = base + ci * CHUNK
        pltpu.sync_copy(x0_hbm.at[pl.ds(off, CHUNK)], x0_v)
        pltpu.sync_copy(x1_hbm.at[pl.ds(off, CHUNK)], x1_v)
        pltpu.sync_copy(x2_hbm.at[pl.ds(off, CHUNK)], x2_v)

        def vec_body(vi, _):
            s0 = vi * (L * UNROLL)
            for u in range(UNROLL):
                s = s0 + u * L
                x0 = x0_v[pl.ds(s, L)]
                x1 = x1_v[pl.ds(s, L)]
                x2 = x2_v[pl.ds(s, L)]
                i0 = _searchsorted(bp0_v, x0, pre0)
                i1 = _searchsorted(bp1_v, x1, pre1)
                i2 = _searchsorted(bp2_v, x2, pre2)
                idx_v[pl.ds(s, L)] = (i0 * G + i1) * G + i2
            return 0

        lax.fori_loop(0, CHUNK // (L * UNROLL), vec_body, 0)
        pltpu.async_copy(vals_hbm.at[idx_v], res_v, sem).wait()
        pltpu.sync_copy(res_v, out_hbm.at[pl.ds(off, CHUNK)])
        return 0

    lax.fori_loop(0, NCHUNK, chunk_body, 0)


@jax.jit
def _run(x0, x1, x2, bp_0, bp_1, bp_2, values_flat):
    mesh = plsc.VectorSubcoreMesh(core_axis_name="c", subcore_axis_name="s")
    k = pl.kernel(
        _body,
        out_type=jax.ShapeDtypeStruct((N,), jnp.float32),
        mesh=mesh,
        compiler_params=pltpu.CompilerParams(needs_layout_passes=False),
        scratch_types=[
            pltpu.VMEM((CHUNK,), jnp.float32),
            pltpu.VMEM((CHUNK,), jnp.float32),
            pltpu.VMEM((CHUNK,), jnp.float32),
            pltpu.VMEM((G,), jnp.float32),
            pltpu.VMEM((G,), jnp.float32),
            pltpu.VMEM((G,), jnp.float32),
            pltpu.VMEM((CHUNK,), jnp.int32),
            pltpu.VMEM((CHUNK,), jnp.float32),
            pltpu.SemaphoreType.DMA,
        ],
    )
    return k(x0, x1, x2, bp_0, bp_1, bp_2, values_flat)


def kernel(x0, x1, x2, bp_0, bp_1, bp_2, values):
    return _run(x0, x1, x2, bp_0, bp_1, bp_2, values.reshape(-1))
